# Initial kernel scaffold; baseline (speedup 1.0000x reference)
import jax
import jax.numpy as jnp
from jax.experimental import pallas as pl


def kernel(x, edge_index, W1, b1, W2, b2):
    raise NotImplementedError("write your pallas kernel here")


# speedup vs baseline: 22.0435x; 22.0435x over previous
import functools

import jax
import jax.numpy as jnp
from jax import lax
from jax.experimental import pallas as pl
from jax.experimental.pallas import tpu as pltpu
from jax.experimental.pallas import tpu_sc as plsc

F32 = jnp.float32

NC = 2
NS = 16
NW = NC * NS
CH = 128


def _sc_mesh():
    return plsc.VectorSubcoreMesh(core_axis_name="c", subcore_axis_name="s")


def _sc_params():
    return pltpu.CompilerParams(use_tc_tiling_on_sc=False)


def _zero_vmem(ref, nrows, d):
    zcols = d // 16

    def zrow(i, _):
        for k in range(zcols):
            ref[i, pl.ds(k * 16, 16)] = jnp.zeros((16,), F32)
        return _

    lax.fori_loop(0, nrows, zrow, None)


def _make_deg_kernel(n_pad, rows):
    sl = n_pad // NS

    @functools.partial(
        pl.kernel,
        out_type=jax.ShapeDtypeStruct((NC * n_pad,), F32),
        mesh=_sc_mesh(),
        compiler_params=_sc_params(),
        scratch_types=[
            pltpu.VMEM((rows * CH,), jnp.int32),
            pltpu.VMEM((rows * CH,), F32),
            pltpu.VMEM((1, sl), F32),
            pltpu.VMEM_SHARED((n_pad,), F32),
        ],
    )
    def deg_kernel(dst_hbm, out_hbm, dst_v, ones_v, stage_v, acc):
        c = lax.axis_index("c")
        s = lax.axis_index("s")
        wid = c * NS + s

        def oinit(i, _):
            ones_v[pl.ds(i * 16, 16)] = jnp.ones((16,), F32)
            return _

        lax.fori_loop(0, rows * CH // 16, oinit, None)
        _zero_vmem(stage_v, 1, sl)
        pltpu.sync_copy(stage_v.at[0], acc.at[pl.ds(s * sl, sl)])
        plsc.subcore_barrier()

        pltpu.sync_copy(dst_hbm.at[wid], dst_v)
        pltpu.sync_copy(ones_v, acc.at[dst_v], add=True)
        plsc.subcore_barrier()

        pltpu.sync_copy(acc.at[pl.ds(s * sl, sl)], stage_v.at[0])
        pltpu.sync_copy(stage_v.at[0], out_hbm.at[pl.ds(c * n_pad + s * sl, sl)])

    return deg_kernel


def _make_agg_kernel(n_pad, rows, d):
    sl = n_pad // NS
    drows = max(d // NS, 1)

    @functools.partial(
        pl.kernel,
        out_type=jax.ShapeDtypeStruct((NC, d, n_pad), F32),
        mesh=_sc_mesh(),
        compiler_params=_sc_params(),
        scratch_types=[
            pltpu.VMEM((rows * CH,), jnp.int32),
            pltpu.VMEM((rows * CH,), jnp.int32),
            pltpu.VMEM((rows * CH,), F32),
            pltpu.VMEM((d, sl), F32),
            pltpu.VMEM((drows, n_pad), F32),
            pltpu.VMEM_SHARED((d, n_pad), F32),
            pltpu.VMEM_SHARED((d, n_pad), F32),
        ],
    )
    def agg_kernel(gT_hbm, src_hbm, dst_hbm, out_hbm,
                   src_v, dst_v, col_v, stage_v, gstage_v, gT_s, accT):
        c = lax.axis_index("c")
        s = lax.axis_index("s")
        wid = c * NS + s

        @pl.when(s * drows < d)
        def _():
            pltpu.sync_copy(gT_hbm.at[pl.ds(s * drows, drows)], gstage_v)
            pltpu.sync_copy(gstage_v, gT_s.at[pl.ds(s * drows, drows)])

        _zero_vmem(stage_v, d, sl)
        pltpu.sync_copy(stage_v, accT.at[:, pl.ds(s * sl, sl)])
        plsc.subcore_barrier()

        pltpu.sync_copy(src_hbm.at[wid], src_v)
        pltpu.sync_copy(dst_hbm.at[wid], dst_v)

        for j in range(d):
            pltpu.sync_copy(gT_s.at[j].at[src_v], col_v)
            pltpu.sync_copy(col_v, accT.at[j].at[dst_v], add=True)
        plsc.subcore_barrier()

        pltpu.sync_copy(accT.at[:, pl.ds(s * sl, sl)], stage_v)
        pltpu.sync_copy(stage_v, out_hbm.at[c, :, pl.ds(s * sl, sl)])

    return agg_kernel


def _tc_layer1(x_pad, w1, deg2, n_pad):
    f = x_pad.shape[1]
    h = w1.shape[1]
    br = 2048
    grid = n_pad // br

    def body(x_ref, w_ref, deg_ref, g_ref, dinv_ref):
        dinv = lax.rsqrt(deg_ref[0] + deg_ref[1] + 1.0)[None, :]
        hm = jnp.dot(x_ref[...], w_ref[...], preferred_element_type=F32)
        g_ref[...] = hm.T * dinv
        dinv_ref[...] = dinv

    return pl.pallas_call(
        body,
        grid=(grid,),
        in_specs=[
            pl.BlockSpec((br, f), lambda i: (i, 0)),
            pl.BlockSpec((f, h), lambda i: (0, 0)),
            pl.BlockSpec((NC, br), lambda i: (0, i)),
        ],
        out_specs=[
            pl.BlockSpec((h, br), lambda i: (0, i)),
            pl.BlockSpec((1, br), lambda i: (0, i)),
        ],
        out_shape=[
            jax.ShapeDtypeStruct((h, n_pad), F32),
            jax.ShapeDtypeStruct((1, n_pad), F32),
        ],
    )(x_pad, w1, deg2)


def _tc_layer2(p, g1T, dinvT, b1c, w2t, n_pad):
    h = g1T.shape[0]
    o = w2t.shape[0]
    br = 2048
    grid = n_pad // br

    def body(p_ref, g1_ref, dinv_ref, b1_ref, w2t_ref, g2_ref):
        dinv = dinv_ref[...]
        pre = dinv * (p_ref[0] + p_ref[1] + g1_ref[...]) + b1_ref[...]
        a1t = jnp.maximum(pre, 0.0)
        h2t = jnp.dot(w2t_ref[...], a1t, preferred_element_type=F32)
        g2_ref[...] = h2t * dinv

    return pl.pallas_call(
        body,
        grid=(grid,),
        in_specs=[
            pl.BlockSpec((NC, h, br), lambda i: (0, 0, i)),
            pl.BlockSpec((h, br), lambda i: (0, i)),
            pl.BlockSpec((1, br), lambda i: (0, i)),
            pl.BlockSpec((h, 1), lambda i: (0, 0)),
            pl.BlockSpec((o, h), lambda i: (0, 0)),
        ],
        out_specs=pl.BlockSpec((o, br), lambda i: (0, i)),
        out_shape=jax.ShapeDtypeStruct((o, n_pad), F32),
    )(p, g1T, dinvT, b1c, w2t)


def _tc_final(q, g2T, dinvT, b2c, n_pad):
    o = g2T.shape[0]
    br = 2048
    grid = n_pad // br

    def body(q_ref, g2_ref, dinv_ref, b2_ref, out_ref):
        outt = (
            dinv_ref[...] * (q_ref[0] + q_ref[1] + g2_ref[...]) + b2_ref[...]
        )
        out_ref[...] = outt.T

    return pl.pallas_call(
        body,
        grid=(grid,),
        in_specs=[
            pl.BlockSpec((NC, o, br), lambda i: (0, 0, i)),
            pl.BlockSpec((o, br), lambda i: (0, i)),
            pl.BlockSpec((1, br), lambda i: (0, i)),
            pl.BlockSpec((o, 1), lambda i: (0, 0)),
        ],
        out_specs=pl.BlockSpec((br, o), lambda i: (i, 0)),
        out_shape=jax.ShapeDtypeStruct((n_pad, o), F32),
    )(q, g2T, dinvT, b2c)


def kernel(x, edge_index, W1, b1, W2, b2):
    n, f = x.shape
    e = edge_index.shape[1]
    h = W1.shape[1]
    o = W2.shape[1]

    n_pad = ((n + 2048) // 2048) * 2048

    ec = e // NW
    rows = (ec + CH - 1) // CH
    ecp = rows * CH
    padn = ecp - ec

    src_t = edge_index[0].reshape(NW, ec)
    dst_t = edge_index[1].reshape(NW, ec)
    if padn:
        pad_idx = n + (jnp.arange(padn, dtype=jnp.int32) % (n_pad - n))
        pads = jnp.broadcast_to(pad_idx, (NW, padn))
        src_t = jnp.concatenate([src_t, pads], axis=1)
        dst_t = jnp.concatenate([dst_t, pads], axis=1)
    src3 = src_t
    dst3 = dst_t

    x_pad = jnp.pad(x, ((0, n_pad - n), (0, 0)))

    deg2 = _make_deg_kernel(n_pad, rows)(dst3).reshape(NC, n_pad)
    g1T, dinvT = _tc_layer1(x_pad, W1, deg2, n_pad)
    p = _make_agg_kernel(n_pad, rows, h)(g1T, src3, dst3)
    g2T = _tc_layer2(p, g1T, dinvT, b1.reshape(h, 1), W2.T, n_pad)
    q = _make_agg_kernel(n_pad, rows, o)(g2T, src3, dst3)
    out = _tc_final(q, g2T, dinvT, b2.reshape(o, 1), n_pad)
    return out[:n]


# speedup vs baseline: 25.4681x; 1.1554x over previous
import functools

import jax
import jax.numpy as jnp
from jax import lax
from jax.experimental import pallas as pl
from jax.experimental.pallas import tpu as pltpu
from jax.experimental.pallas import tpu_sc as plsc

F32 = jnp.float32

NC = 2
NS = 16
NW = NC * NS
CH = 128


def _sc_mesh():
    return plsc.VectorSubcoreMesh(core_axis_name="c", subcore_axis_name="s")


def _sc_params():
    return pltpu.CompilerParams(use_tc_tiling_on_sc=False)


def _zero_vmem(ref, nrows, d):
    zcols = d // 16

    def zrow(i, _):
        for k in range(zcols):
            ref[i, pl.ds(k * 16, 16)] = jnp.zeros((16,), F32)
        return _

    lax.fori_loop(0, nrows, zrow, None)


def _make_deg_kernel(n_pad, rows):
    sl = n_pad // NS

    @functools.partial(
        pl.kernel,
        out_type=jax.ShapeDtypeStruct((NC * n_pad,), F32),
        mesh=_sc_mesh(),
        compiler_params=_sc_params(),
        scratch_types=[
            pltpu.VMEM((rows * CH,), jnp.int32),
            pltpu.VMEM((rows * CH,), F32),
            pltpu.VMEM((1, sl), F32),
            pltpu.VMEM_SHARED((n_pad,), F32),
        ],
    )
    def deg_kernel(dst_hbm, out_hbm, dst_v, ones_v, stage_v, acc):
        c = lax.axis_index("c")
        s = lax.axis_index("s")
        wid = c * NS + s

        def oinit(i, _):
            ones_v[pl.ds(i * 16, 16)] = jnp.ones((16,), F32)
            return _

        lax.fori_loop(0, rows * CH // 16, oinit, None)
        _zero_vmem(stage_v, 1, sl)
        pltpu.sync_copy(stage_v.at[0], acc.at[pl.ds(s * sl, sl)])
        plsc.subcore_barrier()

        pltpu.sync_copy(dst_hbm.at[wid], dst_v)
        pltpu.sync_copy(ones_v, acc.at[dst_v], add=True)
        plsc.subcore_barrier()

        pltpu.sync_copy(acc.at[pl.ds(s * sl, sl)], stage_v.at[0])
        pltpu.sync_copy(stage_v.at[0], out_hbm.at[pl.ds(c * n_pad + s * sl, sl)])

    return deg_kernel


def _make_agg_kernel(n_pad, rows, d):
    sl = n_pad // NS
    drows = max(d // NS, 1)

    @functools.partial(
        pl.kernel,
        out_type=jax.ShapeDtypeStruct((NC, d, n_pad), F32),
        mesh=_sc_mesh(),
        compiler_params=_sc_params(),
        scratch_types=[
            pltpu.VMEM((rows * CH,), jnp.int32),
            pltpu.VMEM((rows * CH,), jnp.int32),
            pltpu.VMEM((2, rows * CH), F32),
            pltpu.VMEM((d, sl), F32),
            pltpu.VMEM((drows, n_pad), F32),
            pltpu.VMEM_SHARED((d, n_pad), F32),
            pltpu.VMEM_SHARED((d, n_pad), F32),
            pltpu.SemaphoreType.DMA,
            pltpu.SemaphoreType.DMA,
        ],
    )
    def agg_kernel(gT_hbm, src_hbm, dst_hbm, out_hbm,
                   src_v, dst_v, col_v, stage_v, gstage_v, gT_s, accT,
                   gsem, ssem):
        c = lax.axis_index("c")
        s = lax.axis_index("s")
        wid = c * NS + s

        @pl.when(s * drows < d)
        def _():
            pltpu.sync_copy(gT_hbm.at[pl.ds(s * drows, drows)], gstage_v)
            pltpu.sync_copy(gstage_v, gT_s.at[pl.ds(s * drows, drows)])

        _zero_vmem(stage_v, d, sl)
        pltpu.sync_copy(stage_v, accT.at[:, pl.ds(s * sl, sl)])
        plsc.subcore_barrier()

        pltpu.sync_copy(src_hbm.at[wid], src_v)
        pltpu.sync_copy(dst_hbm.at[wid], dst_v)

        scat = [None, None]
        for j in range(d):
            b = j & 1
            if scat[b] is not None:
                scat[b].wait()
            pltpu.async_copy(gT_s.at[j].at[src_v], col_v.at[b], gsem).wait()
            scat[b] = pltpu.async_copy(
                col_v.at[b], accT.at[j].at[dst_v], ssem, add=True)
        for dsc in scat:
            if dsc is not None:
                dsc.wait()
        plsc.subcore_barrier()

        pltpu.sync_copy(accT.at[:, pl.ds(s * sl, sl)], stage_v)
        pltpu.sync_copy(stage_v, out_hbm.at[c, :, pl.ds(s * sl, sl)])

    return agg_kernel


def _tc_layer1(x_pad, w1, deg2, n_pad):
    f = x_pad.shape[1]
    h = w1.shape[1]
    br = 2048
    grid = n_pad // br

    def body(x_ref, w_ref, deg_ref, g_ref, dinv_ref):
        dinv = lax.rsqrt(deg_ref[0] + deg_ref[1] + 1.0)[None, :]
        hm = jnp.dot(x_ref[...], w_ref[...], preferred_element_type=F32)
        g_ref[...] = hm.T * dinv
        dinv_ref[...] = dinv

    return pl.pallas_call(
        body,
        grid=(grid,),
        in_specs=[
            pl.BlockSpec((br, f), lambda i: (i, 0)),
            pl.BlockSpec((f, h), lambda i: (0, 0)),
            pl.BlockSpec((NC, br), lambda i: (0, i)),
        ],
        out_specs=[
            pl.BlockSpec((h, br), lambda i: (0, i)),
            pl.BlockSpec((1, br), lambda i: (0, i)),
        ],
        out_shape=[
            jax.ShapeDtypeStruct((h, n_pad), F32),
            jax.ShapeDtypeStruct((1, n_pad), F32),
        ],
    )(x_pad, w1, deg2)


def _tc_layer2(p, g1T, dinvT, b1c, w2t, n_pad):
    h = g1T.shape[0]
    o = w2t.shape[0]
    br = 2048
    grid = n_pad // br

    def body(p_ref, g1_ref, dinv_ref, b1_ref, w2t_ref, g2_ref):
        dinv = dinv_ref[...]
        pre = dinv * (p_ref[0] + p_ref[1] + g1_ref[...]) + b1_ref[...]
        a1t = jnp.maximum(pre, 0.0)
        h2t = jnp.dot(w2t_ref[...], a1t, preferred_element_type=F32)
        g2_ref[...] = h2t * dinv

    return pl.pallas_call(
        body,
        grid=(grid,),
        in_specs=[
            pl.BlockSpec((NC, h, br), lambda i: (0, 0, i)),
            pl.BlockSpec((h, br), lambda i: (0, i)),
            pl.BlockSpec((1, br), lambda i: (0, i)),
            pl.BlockSpec((h, 1), lambda i: (0, 0)),
            pl.BlockSpec((o, h), lambda i: (0, 0)),
        ],
        out_specs=pl.BlockSpec((o, br), lambda i: (0, i)),
        out_shape=jax.ShapeDtypeStruct((o, n_pad), F32),
    )(p, g1T, dinvT, b1c, w2t)


def _tc_final(q, g2T, dinvT, b2c, n_pad):
    o = g2T.shape[0]
    br = 2048
    grid = n_pad // br

    def body(q_ref, g2_ref, dinv_ref, b2_ref, out_ref):
        outt = (
            dinv_ref[...] * (q_ref[0] + q_ref[1] + g2_ref[...]) + b2_ref[...]
        )
        out_ref[...] = outt.T

    return pl.pallas_call(
        body,
        grid=(grid,),
        in_specs=[
            pl.BlockSpec((NC, o, br), lambda i: (0, 0, i)),
            pl.BlockSpec((o, br), lambda i: (0, i)),
            pl.BlockSpec((1, br), lambda i: (0, i)),
            pl.BlockSpec((o, 1), lambda i: (0, 0)),
        ],
        out_specs=pl.BlockSpec((br, o), lambda i: (i, 0)),
        out_shape=jax.ShapeDtypeStruct((n_pad, o), F32),
    )(q, g2T, dinvT, b2c)


def kernel(x, edge_index, W1, b1, W2, b2):
    n, f = x.shape
    e = edge_index.shape[1]
    h = W1.shape[1]
    o = W2.shape[1]

    n_pad = ((n + 2048) // 2048) * 2048

    ec = e // NW
    rows = (ec + CH - 1) // CH
    ecp = rows * CH
    padn = ecp - ec

    src_t = edge_index[0].reshape(NW, ec)
    dst_t = edge_index[1].reshape(NW, ec)
    if padn:
        pad_idx = n + (jnp.arange(padn, dtype=jnp.int32) % (n_pad - n))
        pads = jnp.broadcast_to(pad_idx, (NW, padn))
        src_t = jnp.concatenate([src_t, pads], axis=1)
        dst_t = jnp.concatenate([dst_t, pads], axis=1)
    src3 = src_t
    dst3 = dst_t

    x_pad = jnp.pad(x, ((0, n_pad - n), (0, 0)))

    deg2 = _make_deg_kernel(n_pad, rows)(dst3).reshape(NC, n_pad)
    g1T, dinvT = _tc_layer1(x_pad, W1, deg2, n_pad)
    p = _make_agg_kernel(n_pad, rows, h)(g1T, src3, dst3)
    g2T = _tc_layer2(p, g1T, dinvT, b1.reshape(h, 1), W2.T, n_pad)
    q = _make_agg_kernel(n_pad, rows, o)(g2T, src3, dst3)
    out = _tc_final(q, g2T, dinvT, b2.reshape(o, 1), n_pad)
    return out[:n]
